# Initial kernel scaffold; baseline (speedup 1.0000x reference)
import jax
import jax.numpy as jnp
from jax.experimental import pallas as pl


def kernel(x, t, index):
    raise NotImplementedError("write your pallas kernel here")


# speedup vs baseline: 164.1463x; 164.1463x over previous
import functools
import jax
import jax.numpy as jnp
from jax import lax
from jax.experimental import pallas as pl
from jax.experimental.pallas import tpu as pltpu
from jax.experimental.pallas import tpu_sc as plsc

N_NODES = 100000
N_EDGES = 1600000
D = 32
NC = 2
NS = 16
Q = 25000
Q_PAD = 26624
CHUNK = 512
SUB = 128
FCH = 128
NFCH = (Q + FCH - 1) // FCH
CW = 16
ZROWS = Q_PAD // NS


def _body(x_hbm, t_hbm, idx_hbm, splits_hbm, out_hbm,
          acc_sh, cnt_sh,
          t_buf, idx_raw, idx2d, ones_b, zbuf, zcnt, splits_v,
          x_ch, acc_ch, cnt_ch):
    c0 = lax.axis_index("c")
    s0 = lax.axis_index("s")
    iota = lax.iota(jnp.int32, 16)
    zeros16 = jnp.zeros((16,), jnp.int32)

    zf = jnp.zeros((16,), jnp.float32)
    of = jnp.ones((16,), jnp.float32)
    for r in range(SUB):
        for h in range(D // 16):
            zbuf[r, pl.ds(h * 16, 16)] = zf
        zcnt[r, :] = zf[:CW] if CW == 16 else zf
        ones_b[r, :] = of
    pltpu.sync_copy(splits_hbm, splits_v)

    m1 = splits_v[0][0]
    m2 = splits_v[1][0]
    m3 = splits_v[2][0]

    dump = Q + s0

    def one_pass(pp, _):
        qq = 2 * c0 + pp
        e_lo_raw = jnp.where(qq == 0, 0,
                    jnp.where(qq == 1, m1,
                     jnp.where(qq == 2, m2, m3)))
        e_hi_raw = jnp.where(qq == 0, m1,
                    jnp.where(qq == 1, m2,
                     jnp.where(qq == 2, m3, N_EDGES)))
        lo8 = (e_lo_raw // 8) * 8
        hi8 = ((e_hi_raw + 7) // 8) * 8
        node_base = qq * Q

        z0 = s0 * ZROWS
        for j in range(ZROWS // SUB):
            pltpu.sync_copy(zbuf, acc_sh.at[pl.ds(z0 + j * SUB, SUB)])
            pltpu.sync_copy(zcnt, cnt_sh.at[pl.ds(z0 + j * SUB, SUB)])
        plsc.subcore_barrier()

        nc_chunks = (hi8 - lo8 + CHUNK - 1) // CHUNK
        my_chunks = jnp.maximum(0, (nc_chunks - s0 + NS - 1) // NS)

        def do_chunk(i, _):
            k = s0 + i * NS
            p = lo8 + k * CHUNK
            base = jnp.minimum(p, N_EDGES - CHUNK)
            w_hi = jnp.minimum(p + CHUNK, hi8)
            pltpu.sync_copy(t_hbm.at[pl.ds(base, CHUNK)], t_buf)
            pltpu.sync_copy(idx_hbm.at[pl.ds(base, CHUNK)], idx_raw)
            for v in range(CHUNK // 16):
                vec = idx_raw[pl.ds(v * 16, 16)]
                g = base + (v * 16) + iota
                local = vec - node_base
                keep = ((g >= p) & (g < w_hi)
                        & (local >= 0) & (local < Q))
                lidx = jnp.where(keep, local, dump)
                idx2d[v // 8, pl.ds((v % 8) * 16, 16)] = lidx
            for j in range(CHUNK // SUB):
                pltpu.sync_copy(t_buf.at[pl.ds(j * SUB, SUB)],
                                acc_sh.at[idx2d.at[j]], add=True)
                pltpu.sync_copy(ones_b,
                                cnt_sh.at[idx2d.at[j]], add=True)
            return 0

        lax.fori_loop(0, my_chunks, do_chunk, 0)
        plsc.subcore_barrier()

        my_blocks = jnp.maximum(0, (NFCH - s0 + NS - 1) // NS)

        def do_block(i, _):
            kf = s0 + i * NS
            lnb = jnp.minimum(kf * FCH, Q - FCH)
            gnb = node_base + lnb
            pltpu.sync_copy(x_hbm.at[pl.ds(gnb, FCH)], x_ch)
            pltpu.sync_copy(acc_sh.at[pl.ds(lnb, FCH)], acc_ch)
            pltpu.sync_copy(cnt_sh.at[pl.ds(lnb, FCH)], cnt_ch)
            for r in range(FCH):
                cv = cnt_ch[r, :]
                recip = 1.0 / (cv + 1.0)
                for h in range(D // 16):
                    sl = pl.ds(h * 16, 16)
                    acc_ch[r, sl] = (x_ch[r, sl] + acc_ch[r, sl]) * recip
            pltpu.sync_copy(acc_ch, out_hbm.at[pl.ds(gnb, FCH)])
            return 0

        lax.fori_loop(0, my_blocks, do_block, 0)
        plsc.subcore_barrier()
        return 0

    lax.fori_loop(0, 2, one_pass, 0)


@jax.jit
def _run(x2, t2, idx32, splits):
    mesh = plsc.VectorSubcoreMesh(core_axis_name="c", subcore_axis_name="s")
    f = pl.kernel(
        _body,
        out_type=jax.ShapeDtypeStruct((N_NODES, D), jnp.float32),
        mesh=mesh,
        scratch_types=[
            pltpu.VMEM_SHARED((Q_PAD, D), jnp.float32),
            pltpu.VMEM_SHARED((Q_PAD, CW), jnp.float32),
            pltpu.VMEM((CHUNK, D), jnp.float32),
            pltpu.VMEM((CHUNK,), jnp.int32),
            pltpu.VMEM((CHUNK // SUB, SUB), jnp.int32),
            pltpu.VMEM((SUB, CW), jnp.float32),
            pltpu.VMEM((SUB, D), jnp.float32),
            pltpu.VMEM((SUB, CW), jnp.float32),
            pltpu.VMEM((3, 16), jnp.int32),
            pltpu.VMEM((FCH, D), jnp.float32),
            pltpu.VMEM((FCH, D), jnp.float32),
            pltpu.VMEM((FCH, CW), jnp.float32),
        ],
        compiler_params=pltpu.CompilerParams(use_tc_tiling_on_sc=False),
        name="seg_mean_reduce_sc",
    )
    return f(x2, t2, idx32, splits)


def kernel(x, t, index):
    x2 = x.reshape(N_NODES, D)
    t2 = t.reshape(N_EDGES, D)
    idx32 = index.astype(jnp.int32)
    b = jnp.searchsorted(idx32, jnp.array([Q, 2 * Q, 3 * Q], jnp.int32))
    splits = jnp.broadcast_to(b.astype(jnp.int32)[:, None], (3, 16))
    out = _run(x2, t2, idx32, splits)
    return out.reshape(N_NODES, 4, 8)


# speedup vs baseline: 181.9376x; 1.1084x over previous
import functools
import jax
import jax.numpy as jnp
from jax import lax
from jax.experimental import pallas as pl
from jax.experimental.pallas import tpu as pltpu
from jax.experimental.pallas import tpu_sc as plsc

N_NODES = 100000
N_EDGES = 1600000
D = 32
NC = 2
NS = 16
Q = 25000
Q_PAD = 26624
CHUNK = 512
SUB = 128
FCH = 64
NFCH = (Q + FCH - 1) // FCH
CW = 16
ZROWS = Q_PAD // NS
ZB = 64


def _body(x_hbm, t_hbm, idx_hbm, splits_hbm, out_hbm,
          acc_sh, cnt_sh,
          t_buf0, t_buf1, idx_raw0, idx_raw1, idx2d, ones_b, zbuf, zcnt,
          splits_v, x_ch, acc_ch, cnt_ch, lsem0, lsem1, ssem):
    t_bufs = (t_buf0, t_buf1)
    idx_raws = (idx_raw0, idx_raw1)
    load_sems = (lsem0, lsem1, ssem)
    c0 = lax.axis_index("c")
    s0 = lax.axis_index("s")
    iota = lax.iota(jnp.int32, 16)
    zeros16 = jnp.zeros((16,), jnp.int32)

    zf = jnp.zeros((16,), jnp.float32)
    of = jnp.ones((16,), jnp.float32)
    for r in range(ZB):
        for h in range(D // 16):
            zbuf[r, pl.ds(h * 16, 16)] = zf
        zcnt[r, :] = zf
    for r in range(SUB):
        ones_b[r, :] = of
    pltpu.sync_copy(splits_hbm, splits_v)

    m1 = splits_v[0][0]
    m2 = splits_v[1][0]
    m3 = splits_v[2][0]

    dump = Q + s0

    def one_pass(pp, _):
        qq = 2 * c0 + pp
        e_lo_raw = jnp.where(qq == 0, 0,
                    jnp.where(qq == 1, m1,
                     jnp.where(qq == 2, m2, m3)))
        e_hi_raw = jnp.where(qq == 0, m1,
                    jnp.where(qq == 1, m2,
                     jnp.where(qq == 2, m3, N_EDGES)))
        lo8 = (e_lo_raw // 8) * 8
        hi8 = ((e_hi_raw + 7) // 8) * 8
        node_base = qq * Q

        z0 = s0 * ZROWS
        for j in range(ZROWS // ZB):
            pltpu.sync_copy(zbuf, acc_sh.at[pl.ds(z0 + j * ZB, ZB)])
            pltpu.sync_copy(zcnt, cnt_sh.at[pl.ds(z0 + j * ZB, ZB)])
        plsc.subcore_barrier()

        nc_chunks = (hi8 - lo8 + CHUNK - 1) // CHUNK
        my_chunks = jnp.maximum(0, (nc_chunks - s0 + NS - 1) // NS)

        def chunk_base(j):
            p = lo8 + (s0 + j * NS) * CHUNK
            base = jnp.maximum(0, jnp.minimum(p, N_EDGES - CHUNK))
            return p, pl.multiple_of(base, 8)

        def issue_load(j, b):
            _, base = chunk_base(j)
            pltpu.async_copy(t_hbm.at[pl.ds(base, CHUNK)], t_bufs[b],
                             load_sems[b])
            pltpu.async_copy(idx_hbm.at[pl.ds(base, CHUNK)], idx_raws[b],
                             load_sems[b])

        def wait_load(b):
            pltpu.make_async_copy(t_hbm.at[pl.ds(0, CHUNK)], t_bufs[b],
                                  load_sems[b]).wait()
            pltpu.make_async_copy(idx_hbm.at[pl.ds(0, CHUNK)], idx_raws[b],
                                  load_sems[b]).wait()

        issue_load(0, 0)
        n_pairs = (my_chunks + 1) // 2

        def do_pair(i2, _):
            for b in range(2):
                j = 2 * i2 + b
                p, base = chunk_base(j)
                w_hi = jnp.minimum(p + CHUNK, hi8)
                wait_load(b)
                issue_load(j + 1, 1 - b)

                @pl.when(j < my_chunks)
                def _():
                    for v in range(CHUNK // 16):
                        vec = idx_raws[b][pl.ds(v * 16, 16)]
                        g = base + (v * 16) + iota
                        local = vec - node_base
                        keep = ((g >= p) & (g < w_hi)
                                & (local >= 0) & (local < Q))
                        lidx = jnp.where(keep, local, dump)
                        idx2d[v // 8, pl.ds((v % 8) * 16, 16)] = lidx
                    for j2 in range(CHUNK // SUB):
                        pltpu.sync_copy(t_bufs[b].at[pl.ds(j2 * SUB, SUB)],
                                        acc_sh.at[idx2d.at[j2]], add=True)
                        pltpu.sync_copy(ones_b, cnt_sh.at[idx2d.at[j2]],
                                        add=True)
            return 0

        lax.fori_loop(0, n_pairs, do_pair, 0)
        wait_load(0)
        plsc.subcore_barrier()

        my_blocks = jnp.maximum(0, (NFCH - s0 + NS - 1) // NS)

        def do_block(i, _):
            kf = s0 + i * NS
            lnb = jnp.minimum(kf * FCH, Q - FCH)
            gnb = node_base + lnb
            pltpu.sync_copy(x_hbm.at[pl.ds(gnb, FCH)], x_ch)
            pltpu.sync_copy(acc_sh.at[pl.ds(lnb, FCH)], acc_ch)
            pltpu.sync_copy(cnt_sh.at[pl.ds(lnb, FCH)], cnt_ch)
            for r in range(FCH):
                cv = cnt_ch[r, :]
                recip = 1.0 / (cv + 1.0)
                for h in range(D // 16):
                    sl = pl.ds(h * 16, 16)
                    acc_ch[r, sl] = (x_ch[r, sl] + acc_ch[r, sl]) * recip
            pltpu.sync_copy(acc_ch, out_hbm.at[pl.ds(gnb, FCH)])
            return 0

        lax.fori_loop(0, my_blocks, do_block, 0)
        plsc.subcore_barrier()
        return 0

    lax.fori_loop(0, 2, one_pass, 0)


@jax.jit
def _run(x2, t2, idx32, splits):
    mesh = plsc.VectorSubcoreMesh(core_axis_name="c", subcore_axis_name="s")
    f = pl.kernel(
        _body,
        out_type=jax.ShapeDtypeStruct((N_NODES, D), jnp.float32),
        mesh=mesh,
        scratch_types=[
            pltpu.VMEM_SHARED((Q_PAD, D), jnp.float32),
            pltpu.VMEM_SHARED((Q_PAD, CW), jnp.float32),
            pltpu.VMEM((CHUNK, D), jnp.float32),
            pltpu.VMEM((CHUNK, D), jnp.float32),
            pltpu.VMEM((CHUNK,), jnp.int32),
            pltpu.VMEM((CHUNK,), jnp.int32),
            pltpu.VMEM((CHUNK // SUB, SUB), jnp.int32),
            pltpu.VMEM((SUB, CW), jnp.float32),
            pltpu.VMEM((ZB, D), jnp.float32),
            pltpu.VMEM((ZB, CW), jnp.float32),
            pltpu.VMEM((3, 16), jnp.int32),
            pltpu.VMEM((FCH, D), jnp.float32),
            pltpu.VMEM((FCH, D), jnp.float32),
            pltpu.VMEM((FCH, CW), jnp.float32),
            pltpu.SemaphoreType.DMA,
            pltpu.SemaphoreType.DMA,
            pltpu.SemaphoreType.DMA,
        ],
        compiler_params=pltpu.CompilerParams(use_tc_tiling_on_sc=False),
        name="seg_mean_reduce_sc",
    )
    return f(x2, t2, idx32, splits)


def kernel(x, t, index):
    x2 = x.reshape(N_NODES, D)
    t2 = t.reshape(N_EDGES, D)
    idx32 = index.astype(jnp.int32)
    b = jnp.searchsorted(idx32, jnp.array([Q, 2 * Q, 3 * Q], jnp.int32))
    splits = jnp.broadcast_to(b.astype(jnp.int32)[:, None], (3, 16))
    out = _run(x2, t2, idx32, splits)
    return out.reshape(N_NODES, 4, 8)


# speedup vs baseline: 181.9575x; 1.0001x over previous
import functools
import jax
import jax.numpy as jnp
from jax import lax
from jax.experimental import pallas as pl
from jax.experimental.pallas import tpu as pltpu
from jax.experimental.pallas import tpu_sc as plsc

N_NODES = 100000
N_EDGES = 1600000
D = 32
NC = 2
NS = 16
Q = 25000
Q_PAD = 26624
CHUNK = 512
SUB = 128
FCH = 64
NFCH = (Q + FCH - 1) // FCH
CW = 16
ZROWS = Q_PAD // NS
ZB = 64


def _body(x_hbm, t_hbm, idx_hbm, splits_hbm, out_hbm,
          acc_sh, cnt_sh,
          t_buf0, t_buf1, idx_raw0, idx_raw1, idx2d, ones_b, zbuf, zcnt,
          splits_v, x_ch, acc_ch, cnt_ch, lsem0, lsem1, ssem):
    t_bufs = (t_buf0, t_buf1)
    idx_raws = (idx_raw0, idx_raw1)
    load_sems = (lsem0, lsem1, ssem)
    c0 = lax.axis_index("c")
    s0 = lax.axis_index("s")
    iota = lax.iota(jnp.int32, 16)
    zeros16 = jnp.zeros((16,), jnp.int32)

    zf = jnp.zeros((16,), jnp.float32)
    of = jnp.ones((16,), jnp.float32)
    for r in range(ZB):
        for h in range(D // 16):
            zbuf[r, pl.ds(h * 16, 16)] = zf
        zcnt[r, :] = zf
    for r in range(SUB):
        ones_b[r, :] = of
    pltpu.sync_copy(splits_hbm, splits_v)

    m1 = splits_v[0][0]
    m2 = splits_v[1][0]
    m3 = splits_v[2][0]

    dump = Q + s0

    def one_pass(pp, _):
        qq = 2 * c0 + pp
        e_lo_raw = jnp.where(qq == 0, 0,
                    jnp.where(qq == 1, m1,
                     jnp.where(qq == 2, m2, m3)))
        e_hi_raw = jnp.where(qq == 0, m1,
                    jnp.where(qq == 1, m2,
                     jnp.where(qq == 2, m3, N_EDGES)))
        lo8 = (e_lo_raw // 8) * 8
        hi8 = ((e_hi_raw + 7) // 8) * 8
        node_base = qq * Q

        z0 = s0 * ZROWS
        for j in range(ZROWS // ZB):
            pltpu.sync_copy(zbuf, acc_sh.at[pl.ds(z0 + j * ZB, ZB)])
            pltpu.sync_copy(zcnt, cnt_sh.at[pl.ds(z0 + j * ZB, ZB)])
        plsc.subcore_barrier()

        nc_chunks = (hi8 - lo8 + CHUNK - 1) // CHUNK
        my_chunks = jnp.maximum(0, (nc_chunks - s0 + NS - 1) // NS)

        def chunk_base(j):
            p = lo8 + (s0 + j * NS) * CHUNK
            base = jnp.maximum(0, jnp.minimum(p, N_EDGES - CHUNK))
            return p, pl.multiple_of(base, 8)

        def issue_load(j, b):
            _, base = chunk_base(j)
            pltpu.async_copy(t_hbm.at[pl.ds(base, CHUNK)], t_bufs[b],
                             load_sems[b])
            pltpu.async_copy(idx_hbm.at[pl.ds(base, CHUNK)], idx_raws[b],
                             load_sems[b])

        def wait_load(b):
            pltpu.make_async_copy(t_hbm.at[pl.ds(0, CHUNK)], t_bufs[b],
                                  load_sems[b]).wait()
            pltpu.make_async_copy(idx_hbm.at[pl.ds(0, CHUNK)], idx_raws[b],
                                  load_sems[b]).wait()

        issue_load(0, 0)
        n_pairs = (my_chunks + 1) // 2

        def do_pair(i2, _):
            for b in range(2):
                j = 2 * i2 + b
                p, base = chunk_base(j)
                w_hi = jnp.minimum(p + CHUNK, hi8)
                wait_load(b)
                issue_load(j + 1, 1 - b)

                @pl.when(j < my_chunks)
                def _():
                    for v in range(CHUNK // 16):
                        vec = idx_raws[b][pl.ds(v * 16, 16)]
                        g = base + (v * 16) + iota
                        local = vec - node_base
                        keep = ((g >= p) & (g < w_hi)
                                & (local >= 0) & (local < Q))
                        lidx = jnp.where(keep, local, dump)
                        idx2d[v // 8, pl.ds((v % 8) * 16, 16)] = lidx
                    for j2 in range(CHUNK // SUB):
                        pltpu.sync_copy(t_bufs[b].at[pl.ds(j2 * SUB, SUB)],
                                        acc_sh.at[idx2d.at[j2]], add=True)
                        pltpu.sync_copy(ones_b, cnt_sh.at[idx2d.at[j2]],
                                        add=True)
            return 0

        lax.fori_loop(0, n_pairs, do_pair, 0)
        wait_load(0)
        plsc.subcore_barrier()

        my_blocks = jnp.maximum(0, (NFCH - s0 + NS - 1) // NS)

        def do_block(i, _):
            kf = s0 + i * NS
            lnb = jnp.minimum(kf * FCH, Q - FCH)
            gnb = node_base + lnb
            pltpu.sync_copy(x_hbm.at[pl.ds(gnb, FCH)], x_ch)
            pltpu.sync_copy(acc_sh.at[pl.ds(lnb, FCH)], acc_ch)
            pltpu.sync_copy(cnt_sh.at[pl.ds(lnb, FCH)], cnt_ch)
            for r in range(FCH):
                cv = cnt_ch[r, :]
                recip = 1.0 / (cv + 1.0)
                for h in range(D // 16):
                    sl = pl.ds(h * 16, 16)
                    acc_ch[r, sl] = (x_ch[r, sl] + acc_ch[r, sl]) * recip
            pltpu.sync_copy(acc_ch, out_hbm.at[pl.ds(gnb, FCH)])
            return 0

        lax.fori_loop(0, my_blocks, do_block, 0)
        plsc.subcore_barrier()
        return 0

    lax.fori_loop(0, 2, one_pass, 0)


@jax.jit
def _run(x2, t2, idx32, splits):
    mesh = plsc.VectorSubcoreMesh(core_axis_name="c", subcore_axis_name="s")
    f = pl.kernel(
        _body,
        out_type=jax.ShapeDtypeStruct((N_NODES, D), jnp.float32),
        mesh=mesh,
        scratch_types=[
            pltpu.VMEM_SHARED((Q_PAD, D), jnp.float32),
            pltpu.VMEM_SHARED((Q_PAD, CW), jnp.float32),
            pltpu.VMEM((CHUNK, D), jnp.float32),
            pltpu.VMEM((CHUNK, D), jnp.float32),
            pltpu.VMEM((CHUNK,), jnp.int32),
            pltpu.VMEM((CHUNK,), jnp.int32),
            pltpu.VMEM((CHUNK // SUB, SUB), jnp.int32),
            pltpu.VMEM((SUB, CW), jnp.float32),
            pltpu.VMEM((ZB, D), jnp.float32),
            pltpu.VMEM((ZB, CW), jnp.float32),
            pltpu.VMEM((3, 16), jnp.int32),
            pltpu.VMEM((FCH, D), jnp.float32),
            pltpu.VMEM((FCH, D), jnp.float32),
            pltpu.VMEM((FCH, CW), jnp.float32),
            pltpu.SemaphoreType.DMA,
            pltpu.SemaphoreType.DMA,
            pltpu.SemaphoreType.DMA,
        ],
        compiler_params=pltpu.CompilerParams(use_tc_tiling_on_sc=False),
        name="seg_mean_reduce_sc",
    )
    return f(x2, t2, idx32, splits)


def kernel(x, t, index):
    idx32 = index.astype(jnp.int32)
    b = jnp.searchsorted(idx32, jnp.array([Q, 2 * Q, 3 * Q], jnp.int32))
    splits = jnp.broadcast_to(b.astype(jnp.int32)[:, None], (3, 16))
    x2 = x.reshape(N_NODES, D)
    t2 = t.reshape(N_EDGES, D)
    out = _run(x2, t2, idx32, splits)
    return out.reshape(N_NODES, 4, 8)
